# baseline (speedup 1.0000x reference)
import jax
import jax.numpy as jnp
from jax import lax
from jax.experimental import pallas as pl
from jax.experimental.pallas import tpu as pltpu
from jax.experimental.pallas import tpu_sc as plsc

_NUM_FIELDS = 26
_VOCAB = 1000
_EMB_DIM = 50
_EMB_PAD = 128
_HALF = 64
_NPAIR = _NUM_FIELDS // 2
_BATCH = 4096
_NW = 32
_BB = 128
_NBLK = _BATCH // _BB
_UNITS_PER_W = _NPAIR * _NBLK // _NW
_G = 128


def _sc_gather_body(xcat_hbm, table_hbm, out_hbm, xcv, idxb, rows_v,
                    sem, sem2):
    wid = lax.axis_index("s") * 2 + lax.axis_index("c")
    boff = wid * _BB
    pltpu.sync_copy(xcat_hbm.at[pl.ds(boff * _NUM_FIELDS,
                                      _BB * _NUM_FIELDS)], xcv)
    lanes = lax.iota(jnp.int32, 16)
    for f in range(_NUM_FIELDS):
        for k in range(_BB // 16):
            src = (k * 16 + lanes) * _NUM_FIELDS + f
            v = plsc.load_gather(xcv, [src]) + f * _VOCAB
            idxb[pl.ds(f * _BB + k * 16, 16)] = v
    def fire_gathers(p, b):
        return [pltpu.async_copy(
            table_hbm.at[idxb.at[pl.ds((2 * p + h) * _BB, _BB)]],
            rows_v.at[b, h], sem) for h in range(2)]

    def fire_stores(p, b):
        return [pltpu.async_copy(
            rows_v.at[b, h, slice(None), pl.ds(0, _HALF)],
            out_hbm.at[p, pl.ds(boff, _BB), pl.ds(h * _HALF, _HALF)],
            sem2) for h in range(2)]

    store_cps = [None, None]
    prev_g = fire_gathers(0, 0)
    for p in range(1, _NPAIR + 1):
        b = p % 2
        if p < _NPAIR:
            if store_cps[b] is not None:
                for cp in store_cps[b]:
                    cp.wait()
            gcur = fire_gathers(p, b)
        for cp in prev_g:
            cp.wait()
        store_cps[1 - b] = fire_stores(p - 1, 1 - b)
        if p < _NPAIR:
            prev_g = gcur
    for cps in store_cps:
        if cps is not None:
            for cp in cps:
                cp.wait()


def _sc_gather(xcat_flat, table_pad):
    mesh = plsc.VectorSubcoreMesh(core_axis_name="c", subcore_axis_name="s")
    k = pl.kernel(
        _sc_gather_body,
        out_type=jax.ShapeDtypeStruct((_NPAIR, _BATCH, _EMB_PAD),
                                      jnp.float32),
        mesh=mesh,
        scratch_types=[
            pltpu.VMEM((_BB * _NUM_FIELDS,), jnp.int32),
            pltpu.VMEM((_BB * _NUM_FIELDS,), jnp.int32),
            pltpu.VMEM((2, 2, _BB, _EMB_PAD), jnp.float32),
            pltpu.SemaphoreType.DMA,
            pltpu.SemaphoreType.DMA,
        ],
        compiler_params=pltpu.CompilerParams(use_tc_tiling_on_sc=False,
                                             needs_layout_passes=False),
    )
    return k(xcat_flat, table_pad)


def _mlp_body(xc_ref, xn_ref, w1p_ref, w1b_ref, b1_ref, w2_ref, b2_ref,
              w3_ref, b3_ref, w4_ref, b4_ref, out_ref):
    f32, bf16 = jnp.float32, jnp.bfloat16
    h = jnp.dot(xn_ref[...].astype(bf16), w1b_ref[...],
                preferred_element_type=f32)
    x = jnp.concatenate([xc_ref[f] for f in range(_NPAIR)], axis=1)
    h += jnp.dot(x.astype(bf16), w1p_ref[...], preferred_element_type=f32)
    h = jnp.maximum(h + b1_ref[...], 0.0)
    e = jnp.dot(h.astype(bf16), w2_ref[...],
                preferred_element_type=f32) + b2_ref[...]
    h2 = jnp.dot(e.astype(bf16), w3_ref[...],
                 preferred_element_type=f32) + b3_ref[...]
    h2 = jnp.maximum(h2, 0.0)
    out_ref[...] = (jnp.dot(h2.astype(bf16), w4_ref[...],
                            preferred_element_type=f32) + b4_ref[...])


def _mlp(xc3, x_num, w1p, w1b, b1, w2, b2, w3, b3, w4, b4, bm=1024):
    nb = _BATCH // bm
    full2 = lambda shape: pl.BlockSpec(shape, lambda i: (0, 0))
    return pl.pallas_call(
        _mlp_body,
        grid=(nb,),
        in_specs=[
            pl.BlockSpec((_NPAIR, bm, _EMB_PAD), lambda i: (0, i, 0)),
            pl.BlockSpec((bm, x_num.shape[1]), lambda i: (i, 0)),
            pl.BlockSpec(w1p.shape, lambda i: (0, 0)),
            full2(w1b.shape), full2(b1.shape),
            full2(w2.shape), full2(b2.shape),
            full2(w3.shape), full2(b3.shape),
            full2(w4.shape), full2(b4.shape),
        ],
        out_specs=pl.BlockSpec((bm, w4.shape[1]), lambda i: (i, 0)),
        out_shape=jax.ShapeDtypeStruct((_BATCH, w4.shape[1]), jnp.float32),
    )(xc3, x_num, w1p, w1b, b1, w2, b2, w3, b3, w4, b4)


@jax.jit
def kernel(x_cat, x_num, emb, W1, b1, W2, b2, W3, b3, W4, b4):
    table_pad = jnp.pad(
        emb, ((0, 0), (0, 0), (0, _EMB_PAD - _EMB_DIM))
    ).reshape(_NUM_FIELDS * _VOCAB, _EMB_PAD)
    xc3 = _sc_gather(x_cat.reshape(-1), table_pad)
    bf16 = jnp.bfloat16
    w1r = (W1[:_NUM_FIELDS * _EMB_DIM].astype(bf16)
           .reshape(_NUM_FIELDS, _EMB_DIM, 1024))
    w1p = jnp.pad(w1r, ((0, 0), (0, _HALF - _EMB_DIM), (0, 0))
                  ).reshape(_NPAIR * _EMB_PAD, 1024)
    w1b = W1[_NUM_FIELDS * _EMB_DIM:].astype(bf16)
    return _mlp(xc3, x_num, w1p, w1b, b1.reshape(1, -1),
                W2.astype(bf16), b2.reshape(1, -1), W3.astype(bf16),
                b3.reshape(1, -1), W4.astype(bf16), b4.reshape(1, -1))


# speedup vs baseline: 1.1788x; 1.1788x over previous
import jax
import jax.numpy as jnp
from jax import lax
from jax.experimental import pallas as pl
from jax.experimental.pallas import tpu as pltpu
from jax.experimental.pallas import tpu_sc as plsc

_NUM_FIELDS = 26
_VOCAB = 1000
_EMB_DIM = 50
_EMB_PAD = 128
_HALF = 64
_NPAIR = _NUM_FIELDS // 2
_BATCH = 4096
_NW = 32
_BB = 128
_NBLK = _BATCH // _BB
_UNITS_PER_W = _NPAIR * _NBLK // _NW
_G = 128


def _sc_gather_body(xcat_hbm, table_hbm, out_hbm, xcv, idxb, rows_v,
                    sem, sem2):
    wid = lax.axis_index("s") * 2 + lax.axis_index("c")
    boff = wid * _BB
    pltpu.sync_copy(xcat_hbm.at[pl.ds(boff * _NUM_FIELDS,
                                      _BB * _NUM_FIELDS)], xcv)
    lanes = lax.iota(jnp.int32, 16)
    for f in range(_NUM_FIELDS):
        for k in range(_BB // 16):
            src = (k * 16 + lanes) * _NUM_FIELDS + f
            v = plsc.load_gather(xcv, [src]) + f * _VOCAB
            idxb[pl.ds(f * _BB + k * 16, 16)] = v
    def fire_gathers(p, b):
        return [pltpu.async_copy(
            table_hbm.at[idxb.at[pl.ds((2 * p + h) * _BB, _BB)]],
            rows_v.at[b, h], sem) for h in range(2)]

    def fire_stores(p, b):
        return [pltpu.async_copy(
            rows_v.at[b, h, slice(None), pl.ds(0, _HALF)],
            out_hbm.at[p, pl.ds(boff, _BB), pl.ds(h * _HALF, _HALF)],
            sem2) for h in range(2)]

    store_cps = [None, None]
    prev_g = fire_gathers(0, 0)
    for p in range(1, _NPAIR + 1):
        b = p % 2
        if p < _NPAIR:
            if store_cps[b] is not None:
                for cp in store_cps[b]:
                    cp.wait()
            gcur = fire_gathers(p, b)
        for cp in prev_g:
            cp.wait()
        store_cps[1 - b] = fire_stores(p - 1, 1 - b)
        if p < _NPAIR:
            prev_g = gcur
    for cps in store_cps:
        if cps is not None:
            for cp in cps:
                cp.wait()


def _sc_gather(xcat_flat, table_pad):
    mesh = plsc.VectorSubcoreMesh(core_axis_name="c", subcore_axis_name="s")
    k = pl.kernel(
        _sc_gather_body,
        out_type=jax.ShapeDtypeStruct((_NPAIR, _BATCH, _EMB_PAD),
                                      jnp.float32),
        mesh=mesh,
        scratch_types=[
            pltpu.VMEM((_BB * _NUM_FIELDS,), jnp.int32),
            pltpu.VMEM((_BB * _NUM_FIELDS,), jnp.int32),
            pltpu.VMEM((2, 2, _BB, _EMB_PAD), jnp.float32),
            pltpu.SemaphoreType.DMA,
            pltpu.SemaphoreType.DMA,
        ],
        compiler_params=pltpu.CompilerParams(use_tc_tiling_on_sc=False,
                                             needs_layout_passes=False),
    )
    return k(xcat_flat, table_pad)


def _mlp_body(xc_ref, xn_ref, w1p_ref, w1b_ref, b1_ref, w2t_ref, b2_ref,
              w3t_ref, b3_ref, w4t_ref, b4_ref, out_ref):
    f32, bf16 = jnp.float32, jnp.bfloat16
    h = jnp.dot(xn_ref[...].astype(bf16), w1b_ref[...],
                preferred_element_type=f32)
    x = jnp.concatenate([xc_ref[f] for f in range(_NPAIR)], axis=1)
    h += jnp.dot(x.astype(bf16), w1p_ref[...], preferred_element_type=f32)
    h = jnp.maximum(h + b1_ref[...], 0.0)
    et = lax.dot_general(w2t_ref[...], h.astype(bf16),
                         (((1,), (1,)), ((), ())),
                         preferred_element_type=f32) + b2_ref[...]
    h2t = lax.dot_general(w3t_ref[...], et.astype(bf16),
                          (((1,), (0,)), ((), ())),
                          preferred_element_type=f32) + b3_ref[...]
    h2t = jnp.maximum(h2t, 0.0)
    out_ref[...] = (lax.dot_general(w4t_ref[...], h2t.astype(bf16),
                                    (((1,), (0,)), ((), ())),
                                    preferred_element_type=f32)
                    + b4_ref[...])


def _mlp(xc3, x_num, w1p, w1b, b1, w2, b2, w3, b3, w4, b4, bm=512):
    nb = _BATCH // bm
    full2 = lambda shape: pl.BlockSpec(shape, lambda i: (0, 0))
    return pl.pallas_call(
        _mlp_body,
        grid=(nb,),
        in_specs=[
            pl.BlockSpec((_NPAIR, bm, _EMB_PAD), lambda i: (0, i, 0)),
            pl.BlockSpec((bm, x_num.shape[1]), lambda i: (i, 0)),
            pl.BlockSpec(w1p.shape, lambda i: (0, 0)),
            full2(w1b.shape), full2(b1.shape),
            full2(w2.shape), full2(b2.shape),
            full2(w3.shape), full2(b3.shape),
            full2(w4.shape), full2(b4.shape),
        ],
        out_specs=pl.BlockSpec((w4.shape[0], bm), lambda i: (0, i)),
        out_shape=jax.ShapeDtypeStruct((w4.shape[0], _BATCH), jnp.float32),
    )(xc3, x_num, w1p, w1b, b1, w2, b2, w3, b3, w4, b4)


@jax.jit
def kernel(x_cat, x_num, emb, W1, b1, W2, b2, W3, b3, W4, b4):
    table_pad = jnp.pad(
        emb, ((0, 0), (0, 0), (0, _EMB_PAD - _EMB_DIM))
    ).reshape(_NUM_FIELDS * _VOCAB, _EMB_PAD)
    xc3 = _sc_gather(x_cat.reshape(-1), table_pad)
    bf16 = jnp.bfloat16
    w1r = (W1[:_NUM_FIELDS * _EMB_DIM].astype(bf16)
           .reshape(_NUM_FIELDS, _EMB_DIM, 1024))
    w1p = jnp.pad(w1r, ((0, 0), (0, _HALF - _EMB_DIM), (0, 0))
                  ).reshape(_NPAIR * _EMB_PAD, 1024)
    w1b = W1[_NUM_FIELDS * _EMB_DIM:].astype(bf16)
    out_t = _mlp(xc3, x_num, w1p, w1b, b1.reshape(1, -1),
                 W2.T.astype(bf16), b2.reshape(-1, 1), W3.T.astype(bf16),
                 b3.reshape(-1, 1), W4.T.astype(bf16), b4.reshape(-1, 1))
    return out_t.T


# speedup vs baseline: 1.2071x; 1.0240x over previous
import jax
import jax.numpy as jnp
from jax import lax
from jax.experimental import pallas as pl
from jax.experimental.pallas import tpu as pltpu
from jax.experimental.pallas import tpu_sc as plsc

_NUM_FIELDS = 26
_VOCAB = 1000
_EMB_DIM = 50
_EMB_PAD = 128
_HALF = 64
_NPAIR = _NUM_FIELDS // 2
_BATCH = 4096
_NW = 32
_BB = 128
_NBLK = _BATCH // _BB
_UNITS_PER_W = _NPAIR * _NBLK // _NW
_G = 128


def _sc_gather_body(xcat_hbm, table_hbm, out_hbm, xcv, idxb, rows_v,
                    sem, sem2):
    wid = lax.axis_index("s") * 2 + lax.axis_index("c")
    boff = wid * _BB
    pltpu.sync_copy(xcat_hbm.at[slice(None), pl.ds(boff, _BB)], xcv)
    for f in range(_NUM_FIELDS):
        for k in range(_BB // 16):
            v = xcv[f, pl.ds(k * 16, 16)] + f * _VOCAB
            idxb[pl.ds(f * _BB + k * 16, 16)] = v
    def fire_gathers(p, b):
        return [pltpu.async_copy(
            table_hbm.at[idxb.at[pl.ds((2 * p + h) * _BB, _BB)]],
            rows_v.at[b, h], sem) for h in range(2)]

    def fire_stores(p, b):
        return [pltpu.async_copy(
            rows_v.at[b, h, slice(None), pl.ds(0, _HALF)],
            out_hbm.at[p, pl.ds(boff, _BB), pl.ds(h * _HALF, _HALF)],
            sem2) for h in range(2)]

    store_cps = [None, None]
    prev_g = fire_gathers(0, 0)
    for p in range(1, _NPAIR + 1):
        b = p % 2
        if p < _NPAIR:
            if store_cps[b] is not None:
                for cp in store_cps[b]:
                    cp.wait()
            gcur = fire_gathers(p, b)
        for cp in prev_g:
            cp.wait()
        store_cps[1 - b] = fire_stores(p - 1, 1 - b)
        if p < _NPAIR:
            prev_g = gcur
    for cps in store_cps:
        if cps is not None:
            for cp in cps:
                cp.wait()


def _sc_gather(xcat_flat, table_pad):
    mesh = plsc.VectorSubcoreMesh(core_axis_name="c", subcore_axis_name="s")
    k = pl.kernel(
        _sc_gather_body,
        out_type=jax.ShapeDtypeStruct((_NPAIR, _BATCH, _EMB_PAD),
                                      jnp.float32),
        mesh=mesh,
        scratch_types=[
            pltpu.VMEM((_NUM_FIELDS, _BB), jnp.int32),
            pltpu.VMEM((_BB * _NUM_FIELDS,), jnp.int32),
            pltpu.VMEM((2, 2, _BB, _EMB_PAD), jnp.float32),
            pltpu.SemaphoreType.DMA,
            pltpu.SemaphoreType.DMA,
        ],
        compiler_params=pltpu.CompilerParams(use_tc_tiling_on_sc=False,
                                             needs_layout_passes=False),
    )
    return k(xcat_flat, table_pad)


def _mlp_body(xc_ref, xn_ref, w1p_ref, w1b_ref, b1_ref, w2t_ref, b2_ref,
              w3t_ref, b3_ref, w4t_ref, b4_ref, out_ref):
    f32, bf16 = jnp.float32, jnp.bfloat16
    h = lax.dot_general(xn_ref[...].astype(bf16), w1b_ref[...],
                        (((0,), (0,)), ((), ())),
                        preferred_element_type=f32)
    x = jnp.concatenate([xc_ref[f] for f in range(_NPAIR)], axis=1)
    h += jnp.dot(x.astype(bf16), w1p_ref[...], preferred_element_type=f32)
    h = jnp.maximum(h + b1_ref[...], 0.0)
    et = lax.dot_general(w2t_ref[...], h.astype(bf16),
                         (((1,), (1,)), ((), ())),
                         preferred_element_type=f32) + b2_ref[...]
    h2t = lax.dot_general(w3t_ref[...], et.astype(bf16),
                          (((1,), (0,)), ((), ())),
                          preferred_element_type=f32) + b3_ref[...]
    h2t = jnp.maximum(h2t, 0.0)
    out_ref[...] = (lax.dot_general(w4t_ref[...], h2t.astype(bf16),
                                    (((1,), (0,)), ((), ())),
                                    preferred_element_type=f32)
                    + b4_ref[...])


def _mlp(xc3, x_num, w1p, w1b, b1, w2, b2, w3, b3, w4, b4, bm=512):
    nb = _BATCH // bm
    full2 = lambda shape: pl.BlockSpec(shape, lambda i: (0, 0))
    return pl.pallas_call(
        _mlp_body,
        grid=(nb,),
        in_specs=[
            pl.BlockSpec((_NPAIR, bm, _EMB_PAD), lambda i: (0, i, 0)),
            pl.BlockSpec((x_num.shape[0], bm), lambda i: (0, i)),
            pl.BlockSpec(w1p.shape, lambda i: (0, 0)),
            full2(w1b.shape), full2(b1.shape),
            full2(w2.shape), full2(b2.shape),
            full2(w3.shape), full2(b3.shape),
            full2(w4.shape), full2(b4.shape),
        ],
        out_specs=pl.BlockSpec((w4.shape[0], bm), lambda i: (0, i)),
        out_shape=jax.ShapeDtypeStruct((w4.shape[0], _BATCH), jnp.float32),
    )(xc3, x_num, w1p, w1b, b1, w2, b2, w3, b3, w4, b4)


@jax.jit
def kernel(x_cat, x_num, emb, W1, b1, W2, b2, W3, b3, W4, b4):
    table_pad = jnp.pad(
        emb, ((0, 0), (0, 0), (0, _EMB_PAD - _EMB_DIM))
    ).reshape(_NUM_FIELDS * _VOCAB, _EMB_PAD)
    xc3 = _sc_gather(x_cat.T, table_pad)
    bf16 = jnp.bfloat16
    w1r = (W1[:_NUM_FIELDS * _EMB_DIM].astype(bf16)
           .reshape(_NUM_FIELDS, _EMB_DIM, 1024))
    w1p = jnp.pad(w1r, ((0, 0), (0, _HALF - _EMB_DIM), (0, 0))
                  ).reshape(_NPAIR * _EMB_PAD, 1024)
    w1b = W1[_NUM_FIELDS * _EMB_DIM:].astype(bf16)
    out_t = _mlp(xc3, x_num.T, w1p, w1b, b1.reshape(1, -1),
                 W2.T.astype(bf16), b2.reshape(-1, 1), W3.T.astype(bf16),
                 b3.reshape(-1, 1), W4.T.astype(bf16), b4.reshape(-1, 1))
    return out_t.T


# speedup vs baseline: 1.2719x; 1.0537x over previous
import jax
import jax.numpy as jnp
from jax import lax
from jax.experimental import pallas as pl
from jax.experimental.pallas import tpu as pltpu
from jax.experimental.pallas import tpu_sc as plsc

_NUM_FIELDS = 26
_VOCAB = 1000
_EMB_DIM = 50
_EMB_PAD = 128
_HALF = 64
_NPAIR = _NUM_FIELDS // 2
_BATCH = 4096
_NW = 32
_BH = _BATCH // 2
_BB = _BH // _NW


def _sc_gather_body(half, xcat_hbm, table_hbm, out_hbm, xcv, idxb, rows_v,
                    sem, sem2):
    wid = lax.axis_index("s") * 2 + lax.axis_index("c")
    boff = half * _BH + wid * _BB
    loff = wid * _BB
    pltpu.sync_copy(xcat_hbm.at[slice(None), pl.ds(boff, _BB)], xcv)
    for f in range(_NUM_FIELDS):
        for k in range(_BB // 16):
            v = xcv[f, pl.ds(k * 16, 16)] + f * _VOCAB
            idxb[pl.ds(f * _BB + k * 16, 16)] = v
    def fire_gathers(p, b):
        return [pltpu.async_copy(
            table_hbm.at[idxb.at[pl.ds((2 * p + h) * _BB, _BB)]],
            rows_v.at[b, h], sem) for h in range(2)]

    def fire_stores(p, b):
        return [pltpu.async_copy(
            rows_v.at[b, h, slice(None), pl.ds(0, _HALF)],
            out_hbm.at[p, pl.ds(loff, _BB), pl.ds(h * _HALF, _HALF)],
            sem2) for h in range(2)]

    store_cps = [None, None]
    prev_g = fire_gathers(0, 0)
    for p in range(1, _NPAIR + 1):
        b = p % 2
        if p < _NPAIR:
            if store_cps[b] is not None:
                for cp in store_cps[b]:
                    cp.wait()
            gcur = fire_gathers(p, b)
        for cp in prev_g:
            cp.wait()
        store_cps[1 - b] = fire_stores(p - 1, 1 - b)
        if p < _NPAIR:
            prev_g = gcur
    for cps in store_cps:
        if cps is not None:
            for cp in cps:
                cp.wait()


def _sc_gather(xcat_flat, table_pad, half):
    import functools
    mesh = plsc.VectorSubcoreMesh(core_axis_name="c", subcore_axis_name="s")
    k = pl.kernel(
        functools.partial(_sc_gather_body, half),
        out_type=jax.ShapeDtypeStruct((_NPAIR, _BH, _EMB_PAD),
                                      jnp.float32),
        mesh=mesh,
        scratch_types=[
            pltpu.VMEM((_NUM_FIELDS, _BB), jnp.int32),
            pltpu.VMEM((_BB * _NUM_FIELDS,), jnp.int32),
            pltpu.VMEM((2, 2, _BB, _EMB_PAD), jnp.float32),
            pltpu.SemaphoreType.DMA,
            pltpu.SemaphoreType.DMA,
        ],
        compiler_params=pltpu.CompilerParams(use_tc_tiling_on_sc=False,
                                             needs_layout_passes=False),
    )
    return k(xcat_flat, table_pad)


def _mlp_body(xc_ref, xn_ref, w1p_ref, w1b_ref, b1_ref, w2t_ref, b2_ref,
              w3t_ref, b3_ref, w4t_ref, b4_ref, out_ref):
    f32, bf16 = jnp.float32, jnp.bfloat16
    h = lax.dot_general(xn_ref[...].astype(bf16), w1b_ref[...],
                        (((0,), (0,)), ((), ())),
                        preferred_element_type=f32)
    x = jnp.concatenate([xc_ref[f] for f in range(_NPAIR)], axis=1)
    h += jnp.dot(x.astype(bf16), w1p_ref[...], preferred_element_type=f32)
    h = jnp.maximum(h + b1_ref[...], 0.0)
    et = lax.dot_general(w2t_ref[...], h.astype(bf16),
                         (((1,), (1,)), ((), ())),
                         preferred_element_type=f32) + b2_ref[...]
    h2t = lax.dot_general(w3t_ref[...], et.astype(bf16),
                          (((1,), (0,)), ((), ())),
                          preferred_element_type=f32) + b3_ref[...]
    h2t = jnp.maximum(h2t, 0.0)
    out_ref[...] = (lax.dot_general(w4t_ref[...], h2t.astype(bf16),
                                    (((1,), (0,)), ((), ())),
                                    preferred_element_type=f32)
                    + b4_ref[...])


def _mlp_half(xc3, x_num, w1p, w1b, b1, w2, b2, w3, b3, w4, b4, half,
              prev=None, bm=512):
    nb = _BH // bm
    base = half * (_BH // bm)
    full2 = lambda shape: pl.BlockSpec(shape, lambda i: (0, 0))
    body = _mlp_body
    if prev is not None:
        body = lambda *refs: _mlp_body(*refs[:11], refs[-1])
    args = [xc3, x_num, w1p, w1b, b1, w2, b2, w3, b3, w4, b4]
    in_specs = [
        pl.BlockSpec((_NPAIR, bm, _EMB_PAD), lambda i: (0, i, 0)),
        pl.BlockSpec((x_num.shape[0], bm), lambda i: (0, i + base)),
        pl.BlockSpec(w1p.shape, lambda i: (0, 0)),
        full2(w1b.shape), full2(b1.shape),
        full2(w2.shape), full2(b2.shape),
        full2(w3.shape), full2(b3.shape),
        full2(w4.shape), full2(b4.shape),
    ]
    aliases = {}
    if prev is not None:
        args.append(prev)
        in_specs.append(pl.BlockSpec(memory_space=pltpu.MemorySpace.HBM))
        aliases = {11: 0}
    return pl.pallas_call(
        body,
        grid=(nb,),
        in_specs=in_specs,
        out_specs=pl.BlockSpec((w4.shape[0], bm), lambda i: (0, i + base)),
        out_shape=jax.ShapeDtypeStruct((w4.shape[0], _BATCH), jnp.float32),
        input_output_aliases=aliases,
    )(*args)


@jax.jit
def kernel(x_cat, x_num, emb, W1, b1, W2, b2, W3, b3, W4, b4):
    table_pad = jnp.pad(
        emb, ((0, 0), (0, 0), (0, _EMB_PAD - _EMB_DIM))
    ).reshape(_NUM_FIELDS * _VOCAB, _EMB_PAD)
    xcat_t = x_cat.T
    xc_a = _sc_gather(xcat_t, table_pad, 0)
    xc_b = _sc_gather(xcat_t, table_pad, 1)
    bf16 = jnp.bfloat16
    w1r = (W1[:_NUM_FIELDS * _EMB_DIM].astype(bf16)
           .reshape(_NUM_FIELDS, _EMB_DIM, 1024))
    w1p = jnp.pad(w1r, ((0, 0), (0, _HALF - _EMB_DIM), (0, 0))
                  ).reshape(_NPAIR * _EMB_PAD, 1024)
    w1b = W1[_NUM_FIELDS * _EMB_DIM:].astype(bf16)
    xn_t = x_num.T
    ws = (w1p, w1b, b1.reshape(1, -1), W2.T.astype(bf16),
          b2.reshape(-1, 1), W3.T.astype(bf16), b3.reshape(-1, 1),
          W4.T.astype(bf16), b4.reshape(-1, 1))
    out_a = _mlp_half(xc_a, xn_t, *ws, 0)
    out_t = _mlp_half(xc_b, xn_t, *ws, 1, prev=out_a)
    return out_t.T
